# Initial kernel scaffold; baseline (speedup 1.0000x reference)
import jax
import jax.numpy as jnp
from jax.experimental import pallas as pl


def kernel(x, idxarray, W1, W2):
    raise NotImplementedError("write your pallas kernel here")


# speedup vs baseline: 1.1892x; 1.1892x over previous
import functools

import jax
import jax.numpy as jnp
from jax import lax
from jax.experimental import pallas as pl
from jax.experimental.pallas import tpu as pltpu
from jax.experimental.pallas import tpu_sc as plsc

C = 256
N = 10000
K = 16
INV_SQRT5 = 1.0 / (5.0 ** 0.5)

NC = 2
NS = 16
NW = NC * NS
N_PAD = 10240
PER_W = N_PAD // NW
B_OUT = 4
GROWS = B_OUT * K
NBLK = PER_W // B_OUT
LC = C // 16


def _mean_gather_body(table, idxf, out, idx_v, gbuf0, gbuf1, obuf, sem0, sem1):
    wid = lax.axis_index("s") * NC + lax.axis_index("c")
    base = wid * PER_W
    pltpu.sync_copy(idxf.at[pl.ds(base * K, PER_W * K)], idx_v)

    def start_g(b, gbuf, sem):
        pltpu.make_async_copy(
            table.at[idx_v.at[pl.ds(b * GROWS, GROWS)]], gbuf, sem
        ).start()

    def consume(b, gbuf, sem):
        pltpu.make_async_copy(
            table.at[idx_v.at[pl.ds(b * GROWS, GROWS)]], gbuf, sem
        ).wait()
        obase = b * (B_OUT * C)
        for o in range(B_OUT):
            for c in range(LC):
                acc = gbuf[o * K, pl.ds(c * 16, 16)]
                for k in range(1, K):
                    acc = acc + gbuf[o * K + k, pl.ds(c * 16, 16)]
                obuf[pl.ds(obase + o * C + c * 16, 16)] = acc * (1.0 / K)

    start_g(0, gbuf0, sem0)
    start_g(1, gbuf1, sem1)

    def body(i, carry):
        b0 = 2 * i
        consume(b0, gbuf0, sem0)

        @pl.when(b0 + 2 < NBLK)
        def _():
            start_g(b0 + 2, gbuf0, sem0)

        consume(b0 + 1, gbuf1, sem1)

        @pl.when(b0 + 3 < NBLK)
        def _():
            start_g(b0 + 3, gbuf1, sem1)

        return carry

    lax.fori_loop(0, NBLK // 2, body, 0)
    pltpu.sync_copy(obuf, out.at[pl.ds(base * C, PER_W * C)])


@functools.lru_cache(maxsize=1)
def _get_mean_gather():
    return pl.kernel(
        _mean_gather_body,
        out_type=jax.ShapeDtypeStruct((N_PAD * C,), jnp.float32),
        mesh=plsc.VectorSubcoreMesh(
            core_axis_name="c", subcore_axis_name="s", num_cores=NC, num_subcores=NS
        ),
        scratch_types=[
            pltpu.VMEM((PER_W * K,), jnp.int32),
            pltpu.VMEM((GROWS, C), jnp.float32),
            pltpu.VMEM((GROWS, C), jnp.float32),
            pltpu.VMEM((PER_W * C,), jnp.float32),
            pltpu.SemaphoreType.DMA,
            pltpu.SemaphoreType.DMA,
        ],
    )


_NB = 512


def _pre_body(x_ref, w_ref, o_ref):
    g = jax.nn.gelu(x_ref[...] * INV_SQRT5)
    o_ref[...] = lax.dot_general(
        g, w_ref[...], (((0,), (1,)), ((), ())), preferred_element_type=jnp.float32
    )


def _gelu_body(a_ref, o_ref):
    o_ref[...] = jax.nn.gelu(a_ref[...])


def _post_body(a_ref, w_ref, x_ref, o_ref):
    o_ref[...] = (
        lax.dot_general(
            w_ref[...], a_ref[...], (((1,), (1,)), ((), ())),
            preferred_element_type=jnp.float32,
        )
        + x_ref[...]
    )


_pre = pl.pallas_call(
    _pre_body,
    grid=(N_PAD // _NB,),
    in_specs=[
        pl.BlockSpec((C, _NB), lambda i: (0, i)),
        pl.BlockSpec((C, C), lambda i: (0, 0)),
    ],
    out_specs=pl.BlockSpec((_NB, C), lambda i: (i, 0)),
    out_shape=jax.ShapeDtypeStruct((N_PAD, C), jnp.float32),
)

_gelu = pl.pallas_call(
    _gelu_body,
    grid=(N_PAD // _NB,),
    in_specs=[pl.BlockSpec((_NB, C), lambda i: (i, 0))],
    out_specs=pl.BlockSpec((_NB, C), lambda i: (i, 0)),
    out_shape=jax.ShapeDtypeStruct((N_PAD, C), jnp.float32),
)

_post = pl.pallas_call(
    _post_body,
    grid=(N_PAD // _NB,),
    in_specs=[
        pl.BlockSpec((_NB, C), lambda i: (i, 0)),
        pl.BlockSpec((C, C), lambda i: (0, 0)),
        pl.BlockSpec((C, _NB), lambda i: (0, i)),
    ],
    out_specs=pl.BlockSpec((C, _NB), lambda i: (0, i)),
    out_shape=jax.ShapeDtypeStruct((C, N_PAD), jnp.float32),
)


def kernel(x, idxarray, W1, W2):
    x_pad = jnp.pad(x, ((0, 0), (0, N_PAD - N)))
    idx = jnp.pad(idxarray.astype(jnp.int32), ((0, N_PAD - N), (0, 0)))
    idxf = idx.reshape(-1)

    mean_gather = _get_mean_gather()
    t1 = _pre(x_pad, W1)
    a1 = mean_gather(t1, idxf)
    h2 = _gelu(a1.reshape(N_PAD, C))
    a2 = mean_gather(h2, idxf)
    out = _post(a2.reshape(N_PAD, C), W2, x_pad)
    return out[:, :N]


# speedup vs baseline: 1.3689x; 1.1510x over previous
import functools

import jax
import jax.numpy as jnp
from jax import lax
from jax.experimental import pallas as pl
from jax.experimental.pallas import tpu as pltpu
from jax.experimental.pallas import tpu_sc as plsc

C = 256
N = 10000
K = 16
INV_SQRT5 = 1.0 / (5.0 ** 0.5)

NC = 2
NS = 16
NW = NC * NS
N_PAD = 10240
PER_W = N_PAD // NW
B_OUT = 4
GROWS = B_OUT * K
NBLK = PER_W // B_OUT
LC = C // 16


def _mean_gather_body(table, idxf, out, idx_v, gbuf0, gbuf1, obuf, sem0, sem1):
    wid = lax.axis_index("s") * NC + lax.axis_index("c")
    base = wid * PER_W
    pltpu.sync_copy(idxf.at[pl.ds(base * K, PER_W * K)], idx_v)

    def start_g(b, gbuf, sem):
        pltpu.make_async_copy(
            table.at[idx_v.at[pl.ds(b * GROWS, GROWS)]], gbuf, sem
        ).start()

    def consume(b, gbuf, sem):
        pltpu.make_async_copy(
            table.at[idx_v.at[pl.ds(b * GROWS, GROWS)]], gbuf, sem
        ).wait()
        obase = b * (B_OUT * C)
        for o in range(B_OUT):
            for c in range(LC):
                vals = [gbuf[o * K + k, pl.ds(c * 16, 16)] for k in range(K)]
                while len(vals) > 1:
                    vals = [
                        vals[j] + vals[j + 1] for j in range(0, len(vals) - 1, 2)
                    ] + ([vals[-1]] if len(vals) % 2 else [])
                obuf[pl.ds(obase + o * C + c * 16, 16)] = vals[0] * (1.0 / K)

    start_g(0, gbuf0, sem0)
    start_g(1, gbuf1, sem1)

    def body(i, carry):
        b0 = 2 * i
        consume(b0, gbuf0, sem0)

        @pl.when(b0 + 2 < NBLK)
        def _():
            start_g(b0 + 2, gbuf0, sem0)

        consume(b0 + 1, gbuf1, sem1)

        @pl.when(b0 + 3 < NBLK)
        def _():
            start_g(b0 + 3, gbuf1, sem1)

        return carry

    lax.fori_loop(0, NBLK // 2, body, 0)
    pltpu.sync_copy(obuf, out.at[pl.ds(base * C, PER_W * C)])


@functools.lru_cache(maxsize=1)
def _get_mean_gather():
    return pl.kernel(
        _mean_gather_body,
        out_type=jax.ShapeDtypeStruct((N_PAD * C,), jnp.float32),
        mesh=plsc.VectorSubcoreMesh(
            core_axis_name="c", subcore_axis_name="s", num_cores=NC, num_subcores=NS
        ),
        scratch_types=[
            pltpu.VMEM((PER_W * K,), jnp.int32),
            pltpu.VMEM((GROWS, C), jnp.float32),
            pltpu.VMEM((GROWS, C), jnp.float32),
            pltpu.VMEM((PER_W * C,), jnp.float32),
            pltpu.SemaphoreType.DMA,
            pltpu.SemaphoreType.DMA,
        ],
    )


_NB = 512


def _pre_body(x_ref, w_ref, o_ref):
    g = jax.nn.gelu(x_ref[...] * INV_SQRT5)
    o_ref[...] = lax.dot_general(
        g, w_ref[...], (((0,), (1,)), ((), ())), preferred_element_type=jnp.float32
    )


def _gelu_body(a_ref, o_ref):
    o_ref[...] = jax.nn.gelu(a_ref[...])


def _post_body(a_ref, w_ref, x_ref, o_ref):
    o_ref[...] = (
        lax.dot_general(
            w_ref[...], a_ref[...], (((1,), (1,)), ((), ())),
            preferred_element_type=jnp.float32,
        )
        + x_ref[...]
    )


_pre = pl.pallas_call(
    _pre_body,
    grid=(N_PAD // _NB,),
    in_specs=[
        pl.BlockSpec((C, _NB), lambda i: (0, i)),
        pl.BlockSpec((C, C), lambda i: (0, 0)),
    ],
    out_specs=pl.BlockSpec((_NB, C), lambda i: (i, 0)),
    out_shape=jax.ShapeDtypeStruct((N_PAD, C), jnp.float32),
)

_gelu = pl.pallas_call(
    _gelu_body,
    grid=(N_PAD // _NB,),
    in_specs=[pl.BlockSpec((_NB, C), lambda i: (i, 0))],
    out_specs=pl.BlockSpec((_NB, C), lambda i: (i, 0)),
    out_shape=jax.ShapeDtypeStruct((N_PAD, C), jnp.float32),
)

_post = pl.pallas_call(
    _post_body,
    grid=(N_PAD // _NB,),
    in_specs=[
        pl.BlockSpec((_NB, C), lambda i: (i, 0)),
        pl.BlockSpec((C, C), lambda i: (0, 0)),
        pl.BlockSpec((C, _NB), lambda i: (0, i)),
    ],
    out_specs=pl.BlockSpec((C, _NB), lambda i: (0, i)),
    out_shape=jax.ShapeDtypeStruct((C, N_PAD), jnp.float32),
)


def kernel(x, idxarray, W1, W2):
    x_pad = jnp.pad(x, ((0, 0), (0, N_PAD - N)))
    idx = jnp.pad(idxarray.astype(jnp.int32), ((0, N_PAD - N), (0, 0)))
    idxf = idx.reshape(-1)

    mean_gather = _get_mean_gather()
    t1 = _pre(x_pad, W1)
    a1 = mean_gather(t1, idxf)
    h2 = _gelu(a1.reshape(N_PAD, C))
    a2 = mean_gather(h2, idxf)
    out = _post(a2.reshape(N_PAD, C), W2, x_pad)
    return out[:, :N]
